# baseline (speedup 1.0000x reference)
import functools
import math

import jax
import jax.numpy as jnp
from jax.experimental import pallas as pl
from jax.experimental.pallas import tpu as pltpu

H = 1024
NQ = 16
NKV = 4
HD = 128
E = 8
TOPK = 2
I = 768
EPS = 1e-6
S = 2048
BQ = 256
NH = NQ + 2 * NKV


def _qkv_kernel(x_ref, w_ref, inln_ref, qln_ref, kln_ref, out_ref,
                h_scr, cs_scr, sn_scr):
    j = pl.program_id(0)

    @pl.when(j == 0)
    def _():
        x = x_ref[...]
        h_scr[...] = (
            x * jax.lax.rsqrt(jnp.mean(x * x, axis=1, keepdims=True) + EPS)
            * inln_ref[...]
        )
        lane = jax.lax.broadcasted_iota(jnp.int32, (S, HD), 1)
        li = (lane & (HD // 2 - 1)).astype(jnp.float32)
        pos = jax.lax.broadcasted_iota(jnp.int32, (S, HD), 0).astype(jnp.float32)
        inv = jnp.exp(li * (-math.log(10000.0) / (HD // 2)))
        f = pos * inv
        cs_scr[...] = jnp.cos(f)
        sn_scr[...] = jnp.sin(f) * jnp.where(lane < HD // 2, -1.0, 1.0)

    proj = jnp.dot(h_scr[...], w_ref[...], preferred_element_type=jnp.float32)

    @pl.when(j < NQ + NKV)
    def _():
        scale = jnp.where(j < NQ, qln_ref[...], kln_ref[...])
        normed = (
            proj * jax.lax.rsqrt(jnp.mean(proj * proj, axis=1, keepdims=True) + EPS)
            * scale
        )
        rot = pltpu.roll(normed, HD // 2, axis=1)
        out_ref[...] = normed * cs_scr[...] + rot * sn_scr[...]

    @pl.when(j >= NQ + NKV)
    def _():
        out_ref[...] = proj


NG = 4
SG = S // NG


def _attn_kernel(q_ref, k_ref, v_ref, o_ref, *, row0):
    i = pl.program_id(1)
    kvlen = k_ref.shape[0]
    q = q_ref[...]
    k = k_ref[...]
    s = jax.lax.dot_general(
        q, k, (((1,), (1,)), ((), ())), preferred_element_type=jnp.float32
    ) * (1.0 / math.sqrt(HD))
    row = jax.lax.broadcasted_iota(jnp.int32, (BQ, kvlen), 0) + (row0 + i * BQ)
    col = jax.lax.broadcasted_iota(jnp.int32, (BQ, kvlen), 1)
    s = jnp.where(col <= row, s, -1e9)
    m = jnp.max(s, axis=1, keepdims=True)
    p = jnp.exp(s - m)
    o = jnp.dot(p, v_ref[...], preferred_element_type=jnp.float32)
    o_ref[...] = o * (1.0 / jnp.sum(p, axis=1, keepdims=True))


def _oproj_kernel(a0_ref, a1_ref, a2_ref, a3_ref, x_ref, wo_ref, pln_ref,
                  wg_ref, x2_ref, hf_ref, lg_ref):
    wo = wo_ref[...]
    x2 = x_ref[...] + jnp.concatenate(
        [
            jnp.dot(a_ref[...], wo, preferred_element_type=jnp.float32)
            for a_ref in (a0_ref, a1_ref, a2_ref, a3_ref)
        ],
        axis=0,
    )
    x2_ref[...] = x2
    hf = (
        x2 * jax.lax.rsqrt(jnp.mean(x2 * x2, axis=1, keepdims=True) + EPS)
        * pln_ref[...]
    )
    hf_ref[...] = hf
    lg_ref[...] = jnp.dot(hf, wg_ref[...], preferred_element_type=jnp.float32)


def _moe_kernel(hf_ref, lg_ref, gup_ref, down_ref, x2_ref, out_ref, comb_scr):
    e = pl.program_id(0)

    @pl.when(e == 0)
    def _():
        lg = lg_ref[...]
        m = jnp.max(lg, axis=1, keepdims=True)
        p = jnp.exp(lg - m)
        p = p / jnp.sum(p, axis=1, keepdims=True)
        col = jax.lax.broadcasted_iota(jnp.int32, (S, E), 1)
        w1 = jnp.max(p, axis=1, keepdims=True)
        i1 = jnp.min(jnp.where(p == w1, col, E), axis=1, keepdims=True)
        is1 = col == i1
        p2 = jnp.where(is1, -1.0, p)
        w2 = jnp.max(p2, axis=1, keepdims=True)
        i2 = jnp.min(jnp.where(p2 == w2, col, E), axis=1, keepdims=True)
        is2 = col == i2
        denom = w1 + w2
        comb_scr[...] = jnp.where(
            is1, w1 / denom, jnp.where(is2, w2 / denom, 0.0)
        )

    hf = hf_ref[...].astype(jnp.bfloat16)
    gu = jax.lax.dot_general(
        hf, gup_ref[0].astype(jnp.bfloat16), (((1,), (1,)), ((), ())),
        preferred_element_type=jnp.float32,
    )
    gate = gu[:, :I]
    up = gu[:, I:]
    inter = gate * jax.lax.logistic(gate) * up

    col = jax.lax.broadcasted_iota(jnp.int32, (S, E), 1)
    w = jnp.sum(jnp.where(col == e, comb_scr[...], 0.0), axis=1, keepdims=True)
    inter = inter * w

    contrib = jax.lax.dot_general(
        inter.astype(jnp.bfloat16), down_ref[0].astype(jnp.bfloat16),
        (((1,), (1,)), ((), ())),
        preferred_element_type=jnp.float32,
    )

    @pl.when(e == 0)
    def _():
        out_ref[...] = x2_ref[...] + contrib

    @pl.when(e > 0)
    def _():
        out_ref[...] += contrib


def kernel(hidden_states, in_ln, Wq, Wk, Wv, Wo, q_ln, k_ln, post_ln, Wg,
           gate_up_proj, down_proj):
    x = hidden_states[0]
    w_all = jnp.concatenate([Wq, Wk, Wv], axis=1)

    qkv = pl.pallas_call(
        _qkv_kernel,
        grid=(NH,),
        in_specs=[
            pl.BlockSpec((S, H), lambda j: (0, 0)),
            pl.BlockSpec((H, HD), lambda j: (0, j)),
            pl.BlockSpec((1, H), lambda j: (0, 0)),
            pl.BlockSpec((1, HD), lambda j: (0, 0)),
            pl.BlockSpec((1, HD), lambda j: (0, 0)),
        ],
        out_specs=pl.BlockSpec((S, HD), lambda j: (0, j)),
        out_shape=jax.ShapeDtypeStruct((S, NH * HD), jnp.float32),
        scratch_shapes=[
            pltpu.VMEM((S, H), jnp.float32),
            pltpu.VMEM((S, HD), jnp.float32),
            pltpu.VMEM((S, HD), jnp.float32),
        ],
    )(x, w_all, in_ln.reshape(1, H), q_ln.reshape(1, HD), k_ln.reshape(1, HD))

    attn_groups = []
    for g in range(NG):
        kvlen = (g + 1) * SG
        o_g = pl.pallas_call(
            functools.partial(_attn_kernel, row0=g * SG),
            grid=(NQ, SG // BQ),
            in_specs=[
                pl.BlockSpec((BQ, HD), lambda h, i, g=g: (g * SG // BQ + i, h)),
                pl.BlockSpec((kvlen, HD),
                             lambda h, i: (0, NQ + h // (NQ // NKV))),
                pl.BlockSpec((kvlen, HD),
                             lambda h, i: (0, NQ + NKV + h // (NQ // NKV))),
            ],
            out_specs=pl.BlockSpec((BQ, HD), lambda h, i: (i, h)),
            out_shape=jax.ShapeDtypeStruct((SG, NQ * HD), jnp.float32),
        )(qkv, qkv, qkv)
        attn_groups.append(o_g)

    x2, hf, logits = pl.pallas_call(
        _oproj_kernel,
        in_specs=[
            pl.BlockSpec((SG, NQ * HD), lambda: (0, 0)),
            pl.BlockSpec((SG, NQ * HD), lambda: (0, 0)),
            pl.BlockSpec((SG, NQ * HD), lambda: (0, 0)),
            pl.BlockSpec((SG, NQ * HD), lambda: (0, 0)),
            pl.BlockSpec((S, H), lambda: (0, 0)),
            pl.BlockSpec((NQ * HD, H), lambda: (0, 0)),
            pl.BlockSpec((1, H), lambda: (0, 0)),
            pl.BlockSpec((H, E), lambda: (0, 0)),
        ],
        out_specs=[
            pl.BlockSpec((S, H), lambda: (0, 0)),
            pl.BlockSpec((S, H), lambda: (0, 0)),
            pl.BlockSpec((S, E), lambda: (0, 0)),
        ],
        out_shape=[
            jax.ShapeDtypeStruct((S, H), jnp.float32),
            jax.ShapeDtypeStruct((S, H), jnp.float32),
            jax.ShapeDtypeStruct((S, E), jnp.float32),
        ],
        compiler_params=pltpu.CompilerParams(
            vmem_limit_bytes=100 * 1024 * 1024
        ),
    )(*attn_groups, x, Wo, post_ln.reshape(1, H), Wg)

    out = pl.pallas_call(
        _moe_kernel,
        grid=(E,),
        in_specs=[
            pl.BlockSpec((S, H), lambda e: (0, 0)),
            pl.BlockSpec((S, E), lambda e: (0, 0)),
            pl.BlockSpec((1, 2 * I, H), lambda e: (e, 0, 0)),
            pl.BlockSpec((1, H, I), lambda e: (e, 0, 0)),
            pl.BlockSpec((S, H), lambda e: (0, 0)),
        ],
        out_specs=pl.BlockSpec((S, H), lambda e: (0, 0)),
        out_shape=jax.ShapeDtypeStruct((S, H), jnp.float32),
        scratch_shapes=[pltpu.VMEM((S, E), jnp.float32)],
        compiler_params=pltpu.CompilerParams(
            vmem_limit_bytes=100 * 1024 * 1024
        ),
    )(hf, logits, gate_up_proj, down_proj, x2)

    return out[None]


# speedup vs baseline: 1.0053x; 1.0053x over previous
import functools
import math

import jax
import jax.numpy as jnp
from jax.experimental import pallas as pl
from jax.experimental.pallas import tpu as pltpu

H = 1024
NQ = 16
NKV = 4
HD = 128
E = 8
TOPK = 2
I = 768
EPS = 1e-6
S = 2048
BQ = 256
NH = NQ + 2 * NKV


def _qkv_kernel(x_ref, w_ref, inln_ref, qln_ref, kln_ref, out_ref,
                h_scr, cs_scr, sn_scr):
    j = pl.program_id(0)

    @pl.when(j == 0)
    def _():
        x = x_ref[...]
        h_scr[...] = (
            x * jax.lax.rsqrt(jnp.mean(x * x, axis=1, keepdims=True) + EPS)
            * inln_ref[...]
        )
        lane = jax.lax.broadcasted_iota(jnp.int32, (S, HD), 1)
        li = (lane & (HD // 2 - 1)).astype(jnp.float32)
        pos = jax.lax.broadcasted_iota(jnp.int32, (S, HD), 0).astype(jnp.float32)
        inv = jnp.exp(li * (-math.log(10000.0) / (HD // 2)))
        f = pos * inv
        cs_scr[...] = jnp.cos(f)
        sn_scr[...] = jnp.sin(f) * jnp.where(lane < HD // 2, -1.0, 1.0)

    proj = jnp.dot(h_scr[...].astype(jnp.bfloat16), w_ref[...],
                   preferred_element_type=jnp.float32)

    @pl.when(j < NQ + NKV)
    def _():
        scale = jnp.where(j < NQ, qln_ref[...], kln_ref[...])
        normed = (
            proj * jax.lax.rsqrt(jnp.mean(proj * proj, axis=1, keepdims=True) + EPS)
            * scale
        )
        rot = pltpu.roll(normed, HD // 2, axis=1)
        out_ref[...] = (normed * cs_scr[...] + rot * sn_scr[...]).astype(jnp.bfloat16)

    @pl.when(j >= NQ + NKV)
    def _():
        out_ref[...] = proj.astype(jnp.bfloat16)


NG = 4
SG = S // NG


def _attn_kernel(q_ref, k_ref, v_ref, o_ref, *, row0):
    i = pl.program_id(1)
    kvlen = k_ref.shape[0]
    q = q_ref[...]
    k = k_ref[...]
    s = jax.lax.dot_general(
        q, k, (((1,), (1,)), ((), ())), preferred_element_type=jnp.float32
    ) * (1.0 / math.sqrt(HD))
    row = jax.lax.broadcasted_iota(jnp.int32, (BQ, kvlen), 0) + (row0 + i * BQ)
    col = jax.lax.broadcasted_iota(jnp.int32, (BQ, kvlen), 1)
    s = jnp.where(col <= row, s, -1e9)
    m = jnp.max(s, axis=1, keepdims=True)
    p = jnp.exp(s - m)
    o = jnp.dot(p.astype(jnp.bfloat16), v_ref[...],
                preferred_element_type=jnp.float32)
    o_ref[...] = (o * (1.0 / jnp.sum(p, axis=1, keepdims=True))).astype(jnp.bfloat16)


def _oproj_kernel(a0_ref, a1_ref, a2_ref, a3_ref, x_ref, wo_ref, pln_ref,
                  wg_ref, x2_ref, hf_ref, lg_ref):
    wo = wo_ref[...]
    x2 = x_ref[...] + jnp.concatenate(
        [
            jnp.dot(a_ref[...], wo, preferred_element_type=jnp.float32)
            for a_ref in (a0_ref, a1_ref, a2_ref, a3_ref)
        ],
        axis=0,
    )
    x2_ref[...] = x2
    hf = (
        x2 * jax.lax.rsqrt(jnp.mean(x2 * x2, axis=1, keepdims=True) + EPS)
        * pln_ref[...]
    )
    hf_ref[...] = hf.astype(jnp.bfloat16)
    lg_ref[...] = jnp.dot(hf, wg_ref[...], preferred_element_type=jnp.float32)


def _moe_kernel(hf_ref, lg_ref, gup_ref, down_ref, x2_ref, out_ref, comb_scr):
    e = pl.program_id(0)

    @pl.when(e == 0)
    def _():
        lg = lg_ref[...]
        m = jnp.max(lg, axis=1, keepdims=True)
        p = jnp.exp(lg - m)
        p = p / jnp.sum(p, axis=1, keepdims=True)
        col = jax.lax.broadcasted_iota(jnp.int32, (S, E), 1)
        w1 = jnp.max(p, axis=1, keepdims=True)
        i1 = jnp.min(jnp.where(p == w1, col, E), axis=1, keepdims=True)
        is1 = col == i1
        p2 = jnp.where(is1, -1.0, p)
        w2 = jnp.max(p2, axis=1, keepdims=True)
        i2 = jnp.min(jnp.where(p2 == w2, col, E), axis=1, keepdims=True)
        is2 = col == i2
        denom = w1 + w2
        comb_scr[...] = jnp.where(
            is1, w1 / denom, jnp.where(is2, w2 / denom, 0.0)
        )

    hf = hf_ref[...]
    gu = jax.lax.dot_general(
        hf, gup_ref[0], (((1,), (1,)), ((), ())),
        preferred_element_type=jnp.float32,
    )
    gate = gu[:, :I]
    up = gu[:, I:]
    inter = gate * jax.lax.logistic(gate) * up

    col = jax.lax.broadcasted_iota(jnp.int32, (S, E), 1)
    w = jnp.sum(jnp.where(col == e, comb_scr[...], 0.0), axis=1, keepdims=True)
    inter = inter * w

    contrib = jax.lax.dot_general(
        inter.astype(jnp.bfloat16), down_ref[0],
        (((1,), (1,)), ((), ())),
        preferred_element_type=jnp.float32,
    )

    @pl.when(e == 0)
    def _():
        out_ref[...] = x2_ref[...] + contrib

    @pl.when(e > 0)
    def _():
        out_ref[...] += contrib


def kernel(hidden_states, in_ln, Wq, Wk, Wv, Wo, q_ln, k_ln, post_ln, Wg,
           gate_up_proj, down_proj):
    x = hidden_states[0]
    w_all = jnp.concatenate([Wq, Wk, Wv], axis=1).astype(jnp.bfloat16)

    qkv = pl.pallas_call(
        _qkv_kernel,
        grid=(NH,),
        in_specs=[
            pl.BlockSpec((S, H), lambda j: (0, 0)),
            pl.BlockSpec((H, HD), lambda j: (0, j)),
            pl.BlockSpec((1, H), lambda j: (0, 0)),
            pl.BlockSpec((1, HD), lambda j: (0, 0)),
            pl.BlockSpec((1, HD), lambda j: (0, 0)),
        ],
        out_specs=pl.BlockSpec((S, HD), lambda j: (0, j)),
        out_shape=jax.ShapeDtypeStruct((S, NH * HD), jnp.bfloat16),
        scratch_shapes=[
            pltpu.VMEM((S, H), jnp.float32),
            pltpu.VMEM((S, HD), jnp.float32),
            pltpu.VMEM((S, HD), jnp.float32),
        ],
    )(x, w_all, in_ln.reshape(1, H), q_ln.reshape(1, HD), k_ln.reshape(1, HD))

    attn_groups = []
    for g in range(NG):
        kvlen = (g + 1) * SG
        o_g = pl.pallas_call(
            functools.partial(_attn_kernel, row0=g * SG),
            grid=(NQ, SG // BQ),
            in_specs=[
                pl.BlockSpec((BQ, HD), lambda h, i, g=g: (g * SG // BQ + i, h)),
                pl.BlockSpec((kvlen, HD),
                             lambda h, i: (0, NQ + h // (NQ // NKV))),
                pl.BlockSpec((kvlen, HD),
                             lambda h, i: (0, NQ + NKV + h // (NQ // NKV))),
            ],
            out_specs=pl.BlockSpec((BQ, HD), lambda h, i: (i, h)),
            out_shape=jax.ShapeDtypeStruct((SG, NQ * HD), jnp.bfloat16),
        )(qkv, qkv, qkv)
        attn_groups.append(o_g)

    x2, hf, logits = pl.pallas_call(
        _oproj_kernel,
        in_specs=[
            pl.BlockSpec((SG, NQ * HD), lambda: (0, 0)),
            pl.BlockSpec((SG, NQ * HD), lambda: (0, 0)),
            pl.BlockSpec((SG, NQ * HD), lambda: (0, 0)),
            pl.BlockSpec((SG, NQ * HD), lambda: (0, 0)),
            pl.BlockSpec((S, H), lambda: (0, 0)),
            pl.BlockSpec((NQ * HD, H), lambda: (0, 0)),
            pl.BlockSpec((1, H), lambda: (0, 0)),
            pl.BlockSpec((H, E), lambda: (0, 0)),
        ],
        out_specs=[
            pl.BlockSpec((S, H), lambda: (0, 0)),
            pl.BlockSpec((S, H), lambda: (0, 0)),
            pl.BlockSpec((S, E), lambda: (0, 0)),
        ],
        out_shape=[
            jax.ShapeDtypeStruct((S, H), jnp.float32),
            jax.ShapeDtypeStruct((S, H), jnp.bfloat16),
            jax.ShapeDtypeStruct((S, E), jnp.float32),
        ],
        compiler_params=pltpu.CompilerParams(
            vmem_limit_bytes=100 * 1024 * 1024
        ),
    )(*attn_groups, x, Wo.astype(jnp.bfloat16), post_ln.reshape(1, H), Wg)

    out = pl.pallas_call(
        _moe_kernel,
        grid=(E,),
        in_specs=[
            pl.BlockSpec((S, H), lambda e: (0, 0)),
            pl.BlockSpec((S, E), lambda e: (0, 0)),
            pl.BlockSpec((1, 2 * I, H), lambda e: (e, 0, 0)),
            pl.BlockSpec((1, H, I), lambda e: (e, 0, 0)),
            pl.BlockSpec((S, H), lambda e: (0, 0)),
        ],
        out_specs=pl.BlockSpec((S, H), lambda e: (0, 0)),
        out_shape=jax.ShapeDtypeStruct((S, H), jnp.float32),
        scratch_shapes=[pltpu.VMEM((S, E), jnp.float32)],
        compiler_params=pltpu.CompilerParams(
            vmem_limit_bytes=100 * 1024 * 1024
        ),
    )(hf, logits, gate_up_proj.astype(jnp.bfloat16),
      down_proj.astype(jnp.bfloat16), x2)

    return out[None]
